# baseline (speedup 1.0000x reference)
import jax
import jax.numpy as jnp
from jax import lax
from jax.experimental import pallas as pl
from jax.experimental.pallas import tpu as pltpu
from jax.experimental.pallas import tpu_sc as plsc

_NUM_EMBEDDINGS = 1000000
_DIM = 64
_BATCH = 16384
_HIST = 50

_B = _BATCH * _HIST
_NC = 2
_NS = 16
_NW = _NC * _NS
_BPW = _B // _NW
_CB = 8
_C = _CB * _HIST
_NBUF = 4
_NCHUNKS = _BPW // _C
_ROUNDS = _NCHUNKS // _NBUF


def _gather_body(x_hbm, table_hbm, out3_hbm, idx_all,
                 rows0, rows1, rows2, rows3,
                 g0, g1, g2, g3, o0, o1, o2, o3):
    rows = (rows0, rows1, rows2, rows3)
    gsem = (g0, g1, g2, g3)
    osem = (o0, o1, o2, o3)
    wid = lax.axis_index("s") * _NC + lax.axis_index("c")
    base = wid * _BPW
    bbase = wid * (_BPW // _HIST)

    pltpu.sync_copy(x_hbm.at[pl.ds(base, _BPW)], idx_all)

    def start_gather(i, b):
        pltpu.async_copy(table_hbm.at[idx_all.at[pl.ds(i * _C, _C)]],
                         rows[b], gsem[b])

    def wait_gather(i, b):
        pltpu.make_async_copy(table_hbm.at[idx_all.at[pl.ds(i * _C, _C)]],
                              rows[b], gsem[b]).wait()

    def start_wb(i, b):
        for k in range(_CB):
            pltpu.async_copy(rows[b].at[pl.ds(k * _HIST, _HIST)],
                             out3_hbm.at[bbase + i * _CB + k],
                             osem[b])

    def wait_wb(b):
        for k in range(_CB):
            pltpu.make_async_copy(rows[b].at[pl.ds(k * _HIST, _HIST)],
                                  out3_hbm.at[bbase + k],
                                  osem[b]).wait()

    for b in range(_NBUF - 1):
        start_gather(b, b)

    for b in range(_NBUF):
        wait_gather(b, b)
        start_wb(b, b)
        b2 = (b + _NBUF - 1) % _NBUF
        if b == 0:
            start_gather(_NBUF - 1, b2)
        else:
            wait_wb(b2)
            start_gather(b + _NBUF - 1, b2)

    def round_body(r, carry):
        i0 = r * _NBUF
        for b in range(_NBUF):
            i = i0 + b
            wait_gather(i, b)
            start_wb(i, b)
            b2 = (b + _NBUF - 1) % _NBUF
            ip = i + _NBUF - 1

            @pl.when(ip < _NCHUNKS)
            def _():
                wait_wb(b2)
                start_gather(ip, b2)

        return carry

    lax.fori_loop(1, _ROUNDS, round_body, 0)

    for b in range(_NBUF):
        wait_wb(b)


@jax.jit
def kernel(x, weight):
    xf = x.reshape(-1).astype(jnp.int32)
    mesh = plsc.VectorSubcoreMesh(core_axis_name="c", subcore_axis_name="s")
    out = pl.kernel(
        _gather_body,
        out_type=jax.ShapeDtypeStruct((_BATCH, _HIST, _DIM), jnp.float32),
        mesh=mesh,
        scratch_types=[
            pltpu.VMEM((_BPW,), jnp.int32),
            pltpu.VMEM((_C, _DIM), jnp.float32),
            pltpu.VMEM((_C, _DIM), jnp.float32),
            pltpu.VMEM((_C, _DIM), jnp.float32),
            pltpu.VMEM((_C, _DIM), jnp.float32),
            pltpu.SemaphoreType.DMA,
            pltpu.SemaphoreType.DMA,
            pltpu.SemaphoreType.DMA,
            pltpu.SemaphoreType.DMA,
            pltpu.SemaphoreType.DMA,
            pltpu.SemaphoreType.DMA,
            pltpu.SemaphoreType.DMA,
            pltpu.SemaphoreType.DMA,
        ],
        compiler_params=pltpu.CompilerParams(use_tc_tiling_on_sc=False),
    )(xf, weight)
    return out
